# baseline (speedup 1.0000x reference)
import functools

import jax
import jax.numpy as jnp
from jax import lax
from jax.experimental import pallas as pl
from jax.experimental.pallas import tpu as pltpu
from jax.experimental.pallas import tpu_sc as plsc

N = 10000
E = 320000
D = 128
NUM_LABELS = 40

NC = 2
NS = 16
NW = NC * NS
CHUNK = 80
NCHUNK = 128
HALF = NCHUNK // 2
EPW = NCHUNK * CHUNK
E_PAD = NW * EPW - E
N_JUNK = 16
ROWS_A = 624
TAIL = N - NS * ROWS_A


def _agg_body(h_hbm, src_hbm, dst_hbm, zeros_hbm, out_hbm,
              acc_sh, src_v, dst_v, rows_a, rows_b, semz, sema, semb):
    c = lax.axis_index("c")
    s = lax.axis_index("s")
    wid = s * NC + c

    zcp = pltpu.make_async_copy(zeros_hbm, acc_sh.at[pl.ds(s * ROWS_A, ROWS_A)],
                                semz)
    zcp.start()

    zcp.wait()

    @pl.when(s == NS - 1)
    def _():
        pltpu.sync_copy(zeros_hbm.at[pl.ds(0, TAIL)],
                        acc_sh.at[pl.ds(NS * ROWS_A, TAIL)])

    plsc.subcore_barrier()

    def start_gather(i, buf, sem):
        pltpu.make_async_copy(h_hbm.at[src_v.at[i]], buf, sem).start()

    def wait_gather(buf, sem):
        pltpu.make_async_copy(h_hbm.at[pl.ds(0, CHUNK)], buf, sem).wait()

    def scatter_add(i, buf):
        pltpu.sync_copy(buf, acc_sh.at[dst_v.at[i]], add=True)

    @pl.loop(0, 2)
    def _(p):
        base = p * HALF
        pltpu.sync_copy(src_hbm.at[wid, pl.ds(base, HALF)], src_v)
        pltpu.sync_copy(dst_hbm.at[wid, pl.ds(base, HALF)], dst_v)

        @pl.loop(0, HALF)
        def _(i):
            start_gather(i, rows_a, sema)
            wait_gather(rows_a, sema)
            scatter_add(i, rows_a)

    plsc.subcore_barrier()
    pltpu.sync_copy(acc_sh.at[pl.ds(s * ROWS_A, ROWS_A)],
                    out_hbm.at[c, pl.ds(s * ROWS_A, ROWS_A)])

    @pl.when(s == NS - 1)
    def _():
        pltpu.sync_copy(acc_sh.at[pl.ds(NS * ROWS_A, TAIL)],
                        out_hbm.at[c, pl.ds(NS * ROWS_A, TAIL)])


def _sc_aggregate(h, src, dst, zeros):
    mesh = plsc.VectorSubcoreMesh(core_axis_name="c", subcore_axis_name="s")
    kfn = pl.kernel(
        _agg_body,
        out_type=jax.ShapeDtypeStruct((NC, N, D), jnp.float32),
        mesh=mesh,
        scratch_types=[
            pltpu.VMEM_SHARED((N + N_JUNK, D), jnp.float32),
            pltpu.VMEM((HALF, CHUNK), jnp.int32),
            pltpu.VMEM((HALF, CHUNK), jnp.int32),
            pltpu.VMEM((CHUNK, D), jnp.float32),
            pltpu.VMEM((CHUNK, D), jnp.float32),
            pltpu.SemaphoreType.DMA,
            pltpu.SemaphoreType.DMA,
            pltpu.SemaphoreType.DMA,
        ],
    )
    return kfn(h, src, dst, zeros)


BR = 1000


def _mlp_block(parts_ref, h_ref, W1_ref, b1_ref, W2_ref, b2_ref, o_ref):
    z = h_ref[...] + parts_ref[0] + parts_ref[1]
    z = jnp.dot(z, W1_ref[...], preferred_element_type=jnp.float32) + b1_ref[...]
    z = jnp.maximum(z, 0.0)
    o_ref[...] = jnp.dot(z, W2_ref[...], preferred_element_type=jnp.float32) + b2_ref[...]


def _mlp_final_block(parts_ref, h_ref, W1_ref, b1_ref, W2_ref, b2_ref,
                     Wc_ref, bc_ref, o_ref, logits_ref):
    z = h_ref[...] + parts_ref[0] + parts_ref[1]
    z = jnp.dot(z, W1_ref[...], preferred_element_type=jnp.float32) + b1_ref[...]
    z = jnp.maximum(z, 0.0)
    h_out = jnp.dot(z, W2_ref[...], preferred_element_type=jnp.float32) + b2_ref[...]
    o_ref[...] = h_out
    logits_ref[...] = (jnp.dot(h_out, Wc_ref[...], preferred_element_type=jnp.float32)
                       + bc_ref[...])


def _row_spec(block_rows, cols):
    return pl.BlockSpec((block_rows, cols), lambda i: (i, 0))


def _full_spec(shape):
    return pl.BlockSpec(shape, lambda i: tuple(0 for _ in shape))


def _tc_mlp(parts, h, W1, b1, W2, b2):
    return pl.pallas_call(
        _mlp_block,
        grid=(N // BR,),
        in_specs=[
            pl.BlockSpec((NC, BR, D), lambda i: (0, i, 0)),
            _row_spec(BR, D),
            _full_spec((D, D)),
            _full_spec((1, D)),
            _full_spec((D, D)),
            _full_spec((1, D)),
        ],
        out_specs=_row_spec(BR, D),
        out_shape=jax.ShapeDtypeStruct((N, D), jnp.float32),
    )(parts, h, W1, b1.reshape(1, D), W2, b2.reshape(1, D))


def _tc_mlp_final(parts, h, W1, b1, W2, b2, Wc, bc):
    return pl.pallas_call(
        _mlp_final_block,
        grid=(N // BR,),
        in_specs=[
            pl.BlockSpec((NC, BR, D), lambda i: (0, i, 0)),
            _row_spec(BR, D),
            _full_spec((D, D)),
            _full_spec((1, D)),
            _full_spec((D, D)),
            _full_spec((1, D)),
            _full_spec((D, NUM_LABELS)),
            _full_spec((1, NUM_LABELS)),
        ],
        out_specs=[_row_spec(BR, D), _row_spec(BR, NUM_LABELS)],
        out_shape=[jax.ShapeDtypeStruct((N, D), jnp.float32),
                   jax.ShapeDtypeStruct((N, NUM_LABELS), jnp.float32)],
    )(parts, h, W1, b1.reshape(1, D), W2, b2.reshape(1, D),
      Wc, bc.reshape(1, NUM_LABELS))


def kernel(x, edge_index, W1_0, b1_0, W2_0, b2_0, W1_1, b1_1, W2_1, b2_1,
           W1_2, b1_2, W2_2, b2_2, Wc, bc):
    src_pad = jnp.zeros((E_PAD,), jnp.int32)
    dst_pad = N + (jnp.arange(E_PAD, dtype=jnp.int32) % N_JUNK)
    src = jnp.concatenate([edge_index[0], src_pad]).reshape(NW, NCHUNK, CHUNK)
    dst = jnp.concatenate([edge_index[1], dst_pad]).reshape(NW, NCHUNK, CHUNK)
    zeros = jnp.zeros((ROWS_A, D), jnp.float32)

    h = x
    parts = _sc_aggregate(h, src, dst, zeros)
    h = _tc_mlp(parts, h, W1_0, b1_0, W2_0, b2_0)
    parts = _sc_aggregate(h, src, dst, zeros)
    h = _tc_mlp(parts, h, W1_1, b1_1, W2_1, b2_1)
    parts = _sc_aggregate(h, src, dst, zeros)
    h, logits = _tc_mlp_final(parts, h, W1_2, b1_2, W2_2, b2_2, Wc, bc)
    return (h, logits)


# speedup vs baseline: 2.3474x; 2.3474x over previous
import functools

import jax
import jax.numpy as jnp
from jax import lax
from jax.experimental import pallas as pl
from jax.experimental.pallas import tpu as pltpu
from jax.experimental.pallas import tpu_sc as plsc

N = 10000
E = 320000
D = 128
NUM_LABELS = 40

NC = 2
NS = 16
NW = NC * NS
CHUNK = 80
EPW = E // NW
NCHUNK = EPW // CHUNK
ROWS_A = 624
TAIL = N - NS * ROWS_A


def _agg_body(h_hbm, src_hbm, dst_hbm, zeros_hbm, out_hbm,
              acc_sh, src_a, dst_a, src_b, dst_b, rows_a, rows_b,
              semz, sema, semb):
    c = lax.axis_index("c")
    s = lax.axis_index("s")
    wid = s * NC + c

    zcp = pltpu.make_async_copy(zeros_hbm, acc_sh.at[pl.ds(s * ROWS_A, ROWS_A)],
                                semz)
    zcp.start()
    zcp.wait()

    @pl.when(s == NS - 1)
    def _():
        pltpu.sync_copy(zeros_hbm.at[pl.ds(0, TAIL)],
                        acc_sh.at[pl.ds(NS * ROWS_A, TAIL)])

    plsc.subcore_barrier()

    base_w = wid * EPW

    def load_idx(i, sv, dv):
        pltpu.sync_copy(src_hbm.at[pl.ds(base_w + i * CHUNK, CHUNK)], sv)
        pltpu.sync_copy(dst_hbm.at[pl.ds(base_w + i * CHUNK, CHUNK)], dv)

    def start_gather(sv, buf, sem):
        pltpu.make_async_copy(h_hbm.at[sv], buf, sem).start()

    def wait_gather(buf, sem):
        pltpu.make_async_copy(h_hbm.at[pl.ds(0, CHUNK)], buf, sem).wait()

    def scatter_add(dv, buf):
        pltpu.sync_copy(buf, acc_sh.at[dv], add=True)

    @pl.loop(0, NCHUNK - 1, step=2)
    def _(i):
        load_idx(i, src_a, dst_a)
        start_gather(src_a, rows_a, sema)
        load_idx(i + 1, src_b, dst_b)
        start_gather(src_b, rows_b, semb)
        wait_gather(rows_a, sema)
        scatter_add(dst_a, rows_a)
        wait_gather(rows_b, semb)
        scatter_add(dst_b, rows_b)

    load_idx(NCHUNK - 1, src_a, dst_a)
    start_gather(src_a, rows_a, sema)
    wait_gather(rows_a, sema)
    scatter_add(dst_a, rows_a)

    plsc.subcore_barrier()
    pltpu.sync_copy(acc_sh.at[pl.ds(s * ROWS_A, ROWS_A)],
                    out_hbm.at[c, pl.ds(s * ROWS_A, ROWS_A)])

    @pl.when(s == NS - 1)
    def _():
        pltpu.sync_copy(acc_sh.at[pl.ds(NS * ROWS_A, TAIL)],
                        out_hbm.at[c, pl.ds(NS * ROWS_A, TAIL)])


def _sc_aggregate(h, src, dst, zeros):
    mesh = plsc.VectorSubcoreMesh(core_axis_name="c", subcore_axis_name="s")
    kfn = pl.kernel(
        _agg_body,
        out_type=jax.ShapeDtypeStruct((NC, N, D), jnp.float32),
        mesh=mesh,
        scratch_types=[
            pltpu.VMEM_SHARED((N, D), jnp.float32),
            pltpu.VMEM((CHUNK,), jnp.int32),
            pltpu.VMEM((CHUNK,), jnp.int32),
            pltpu.VMEM((CHUNK,), jnp.int32),
            pltpu.VMEM((CHUNK,), jnp.int32),
            pltpu.VMEM((CHUNK, D), jnp.float32),
            pltpu.VMEM((CHUNK, D), jnp.float32),
            pltpu.SemaphoreType.DMA,
            pltpu.SemaphoreType.DMA,
            pltpu.SemaphoreType.DMA,
        ],
    )
    return kfn(h, src, dst, zeros)


BR = 1000


def _mlp_block(parts_ref, h_ref, W1_ref, b1_ref, W2_ref, b2_ref, o_ref):
    z = h_ref[...] + parts_ref[0] + parts_ref[1]
    z = jnp.dot(z, W1_ref[...], preferred_element_type=jnp.float32) + b1_ref[...]
    z = jnp.maximum(z, 0.0)
    o_ref[...] = jnp.dot(z, W2_ref[...], preferred_element_type=jnp.float32) + b2_ref[...]


def _mlp_final_block(parts_ref, h_ref, W1_ref, b1_ref, W2_ref, b2_ref,
                     Wc_ref, bc_ref, o_ref, logits_ref):
    z = h_ref[...] + parts_ref[0] + parts_ref[1]
    z = jnp.dot(z, W1_ref[...], preferred_element_type=jnp.float32) + b1_ref[...]
    z = jnp.maximum(z, 0.0)
    h_out = jnp.dot(z, W2_ref[...], preferred_element_type=jnp.float32) + b2_ref[...]
    o_ref[...] = h_out
    logits_ref[...] = (jnp.dot(h_out, Wc_ref[...], preferred_element_type=jnp.float32)
                       + bc_ref[...])


def _row_spec(block_rows, cols):
    return pl.BlockSpec((block_rows, cols), lambda i: (i, 0))


def _full_spec(shape):
    return pl.BlockSpec(shape, lambda i: tuple(0 for _ in shape))


def _tc_mlp(parts, h, W1, b1, W2, b2):
    return pl.pallas_call(
        _mlp_block,
        grid=(N // BR,),
        in_specs=[
            pl.BlockSpec((NC, BR, D), lambda i: (0, i, 0)),
            _row_spec(BR, D),
            _full_spec((D, D)),
            _full_spec((1, D)),
            _full_spec((D, D)),
            _full_spec((1, D)),
        ],
        out_specs=_row_spec(BR, D),
        out_shape=jax.ShapeDtypeStruct((N, D), jnp.float32),
    )(parts, h, W1, b1.reshape(1, D), W2, b2.reshape(1, D))


def _tc_mlp_final(parts, h, W1, b1, W2, b2, Wc, bc):
    return pl.pallas_call(
        _mlp_final_block,
        grid=(N // BR,),
        in_specs=[
            pl.BlockSpec((NC, BR, D), lambda i: (0, i, 0)),
            _row_spec(BR, D),
            _full_spec((D, D)),
            _full_spec((1, D)),
            _full_spec((D, D)),
            _full_spec((1, D)),
            _full_spec((D, NUM_LABELS)),
            _full_spec((1, NUM_LABELS)),
        ],
        out_specs=[_row_spec(BR, D), _row_spec(BR, NUM_LABELS)],
        out_shape=[jax.ShapeDtypeStruct((N, D), jnp.float32),
                   jax.ShapeDtypeStruct((N, NUM_LABELS), jnp.float32)],
    )(parts, h, W1, b1.reshape(1, D), W2, b2.reshape(1, D),
      Wc, bc.reshape(1, NUM_LABELS))


def kernel(x, edge_index, W1_0, b1_0, W2_0, b2_0, W1_1, b1_1, W2_1, b2_1,
           W1_2, b1_2, W2_2, b2_2, Wc, bc):
    src = edge_index[0]
    dst = edge_index[1]
    zeros = jnp.zeros((ROWS_A, D), jnp.float32)

    h = x
    parts = _sc_aggregate(h, src, dst, zeros)
    h = _tc_mlp(parts, h, W1_0, b1_0, W2_0, b2_0)
    parts = _sc_aggregate(h, src, dst, zeros)
    h = _tc_mlp(parts, h, W1_1, b1_1, W2_1, b2_1)
    parts = _sc_aggregate(h, src, dst, zeros)
    h, logits = _tc_mlp_final(parts, h, W1_2, b1_2, W2_2, b2_2, Wc, bc)
    return (h, logits)
